# baseline (speedup 1.0000x reference)
import functools

import jax
import jax.numpy as jnp
from jax import lax
from jax.experimental import pallas as pl
from jax.experimental.pallas import tpu as pltpu
from jax.experimental.pallas import tpu_sc as plsc

B = 16
N = 4096
D = 16
K = 32
L = 16
CH = N // 2
DELTA_V = 0.1
TWO_DELTA_D = 1.0


def _rsqrt(x):
    i = plsc.bitcast(x, jnp.int32)
    y = plsc.bitcast(jnp.int32(0x5F3759DF) - (i >> 1), jnp.float32)
    for _ in range(3):
        y = y * (1.5 - 0.5 * x * y * y)
    return y


def _sqrt(x):
    return x * _rsqrt(x)


_GDN = lax.GatherDimensionNumbers(
    offset_dims=(), collapsed_slice_dims=(0,), start_index_map=(0,))


def _shuffle(x, idx):
    return lax.gather(x, idx[:, None], dimension_numbers=_GDN,
                      slice_sizes=(1,),
                      mode=lax.GatherScatterMode.PROMISE_IN_BOUNDS)


def _hsum(x, iota):
    for sh in (8, 4, 2, 1):
        x = x + _shuffle(x, iota ^ sh)
    return x


def _sc_body(emb_hbm, lab_hbm, out_hbm,
             emb_v, lab_v, acc_v, part_v, pp_v, mean_v, mt_v, out_v, shared):
    c = lax.axis_index("c")
    s = lax.axis_index("s")
    batch = c * 8 + (s >> 1)
    half = s & 1
    base_pt = batch * N + half * CH

    with jax.named_scope("p0_dma_in"):
        pltpu.sync_copy(emb_hbm.at[pl.ds(base_pt * D, CH * D)], emb_v)
        pltpu.sync_copy(lab_hbm.at[pl.ds(base_pt, CH)], lab_v)

    zeros = jnp.zeros((L,), jnp.float32)
    ones = jnp.ones((L,), jnp.float32)
    iota16 = lax.iota(jnp.int32, L)

    def _zero(k, _):
        acc_v[pl.ds(k * L, L)] = zeros
        return 0
    lax.fori_loop(0, 96, _zero, 0)

    def _segsum(g, _):
        labs = lab_v[pl.ds(g * L, L)]
        for j in range(L):
            o = labs[j] * L
            row = emb_v[pl.ds((g * L + j) * L, L)]
            plsc.addupdate(acc_v.at[pl.ds(o, L)], row)
            plsc.addupdate(acc_v.at[pl.ds(512 + o, L)], ones)
        return 0
    with jax.named_scope("p1_segsum"):
        lax.fori_loop(0, CH // L, _segsum, 0)

    sh_base = s * 1536
    partner = (s ^ 1) * 1536
    pltpu.sync_copy(acc_v.at[pl.ds(0, 1024)], shared.at[pl.ds(sh_base, 1024)])
    plsc.subcore_barrier()
    pltpu.sync_copy(shared.at[pl.ds(partner, 1024)], part_v)

    def _comb(k, _):
        tot = acc_v[pl.ds(k * L, L)] + part_v[pl.ds(k * L, L)]
        cnt = acc_v[pl.ds(512 + k * L, L)] + part_v[pl.ds(512 + k * L, L)]
        acc_v[pl.ds(k * L, L)] = tot
        acc_v[pl.ds(512 + k * L, L)] = cnt
        mean_v[pl.ds(k * L, L)] = tot / jnp.maximum(cnt, 1.0)
        return 0
    with jax.named_scope("p2_combine"):
        lax.fori_loop(0, K, _comb, 0)

    def _pull(g, _):
        labs = lab_v[pl.ds(g * L, L)]
        for j in range(L):
            o = labs[j] * L
            row = emb_v[pl.ds((g * L + j) * L, L)]
            dv = row - mean_v[pl.ds(o, L)]
            sqv = jnp.maximum(_hsum(dv * dv, iota16), 1e-20)
            dl = jnp.maximum(_sqrt(sqv) - DELTA_V, 0.0)
            plsc.addupdate(acc_v.at[pl.ds(1024 + o, L)], dl)
        return 0
    with jax.named_scope("p3_pull"):
        lax.fori_loop(0, CH // L, _pull, 0)

    pltpu.sync_copy(acc_v.at[pl.ds(1024, 512)],
                    shared.at[pl.ds(sh_base + 1024, 512)])
    plsc.subcore_barrier()
    pltpu.sync_copy(shared.at[pl.ds(partner + 1024, 512)], pp_v)

    @pl.when(half == 0)
    def _finalize():
        def _combp(k, _):
            plsc.addupdate(acc_v.at[pl.ds(1024 + k * L, L)],
                           pp_v[pl.ds(k * L, L)])
            return 0
        lax.fori_loop(0, K, _combp, 0)

        diag = iota16 * 17
        cnt0 = plsc.load_gather(acc_v, [512 + diag])
        cnt1 = plsc.load_gather(acc_v, [768 + diag])
        ps0 = plsc.load_gather(acc_v, [1024 + diag])
        ps1 = plsc.load_gather(acc_v, [1280 + diag])
        f0 = jnp.where((cnt0 > 0.0) & (iota16 >= 1), 1.0, 0.0)
        f1 = jnp.where(cnt1 > 0.0, 1.0, 0.0)
        pim0 = ps0 / jnp.maximum(cnt0, 1.0)
        pim1 = ps1 / jnp.maximum(cnt1, 1.0)
        num_inst = _hsum(f0 + f1, iota16)
        instance_pull = _hsum(pim0 * f0 + pim1 * f1, iota16)
        pull_b = jnp.where(num_inst > 0.0,
                           instance_pull / (num_inst + 1e-6), 0.0)

        for k in range(K):
            plsc.store_scatter(mt_v, [iota16 * K + k], mean_v[pl.ds(k * L, L)])
        nsq0 = zeros
        nsq1 = zeros
        for d in range(D):
            t0 = mt_v[pl.ds(d * K, L)]
            t1 = mt_v[pl.ds(d * K + L, L)]
            nsq0 = nsq0 + t0 * t0
            nsq1 = nsq1 + t1 * t1
        rinv0 = jnp.minimum(_rsqrt(jnp.maximum(nsq0, 1e-30)), 1e12) * f0
        rinv1 = jnp.minimum(_rsqrt(jnp.maximum(nsq1, 1e-30)), 1e12) * f1
        for d in range(D):
            mt_v[pl.ds(d * K, L)] = mt_v[pl.ds(d * K, L)] * rinv0
            mt_v[pl.ds(d * K + L, L)] = mt_v[pl.ds(d * K + L, L)] * rinv1

        dsum = zeros
        msum = zeros
        for i in range(1, K):
            acc0 = zeros
            acc1 = zeros
            for d in range(D):
                mid = mt_v[pl.ds(d * K + (i // L) * L, L)][i % L]
                acc0 = acc0 + mid * mt_v[pl.ds(d * K, L)]
                acc1 = acc1 + mid * mt_v[pl.ds(d * K + L, L)]
            sqd0 = jnp.maximum(2.0 - 2.0 * acc0, 1e-20)
            sqd1 = jnp.maximum(2.0 - 2.0 * acc1, 1e-20)
            h0 = jnp.maximum(TWO_DELTA_D - _sqrt(sqd0), 0.0)
            h1 = jnp.maximum(TWO_DELTA_D - _sqrt(sqd1), 0.0)
            cnt_i = acc_v[pl.ds(512 + i * L, L)]
            fi = jnp.where(cnt_i > 0.0, 1.0, 0.0)
            m0 = f0 * fi * jnp.where(iota16 > i, 1.0, 0.0)
            m1 = f1 * fi * jnp.where(iota16 + L > i, 1.0, 0.0)
            dsum = dsum + h0 * m0 + h1 * m1
            msum = msum + m0 + m1
        push_b = jnp.where(num_inst > 1.0,
                           _hsum(dsum, iota16)
                           / (_hsum(msum, iota16) + 1e-6), 0.0)

        out_v[...] = (jnp.where(iota16 == 0, pull_b, 0.0)
                      + jnp.where(iota16 == 1, push_b, 0.0))
        pltpu.sync_copy(out_v, out_hbm.at[pl.ds(batch * L, L)])


@jax.jit
def _sc_losses(emb_flat, lab_flat):
    mesh = plsc.VectorSubcoreMesh(core_axis_name="c", subcore_axis_name="s")
    f = functools.partial(
        pl.kernel,
        mesh=mesh,
        compiler_params=pltpu.CompilerParams(needs_layout_passes=False),
        out_type=jax.ShapeDtypeStruct((B * L,), jnp.float32),
        scratch_types=[
            pltpu.VMEM((CH * D,), jnp.float32),
            pltpu.VMEM((CH,), jnp.int32),
            pltpu.VMEM((1536,), jnp.float32),
            pltpu.VMEM((1024,), jnp.float32),
            pltpu.VMEM((512,), jnp.float32),
            pltpu.VMEM((512,), jnp.float32),
            pltpu.VMEM((512,), jnp.float32),
            pltpu.VMEM((L,), jnp.float32),
            pltpu.VMEM_SHARED((16 * 1536,), jnp.float32),
        ],
    )(_sc_body)
    return f(emb_flat, lab_flat)


def kernel(embeddings, labels):
    emb_flat = embeddings.reshape(-1)
    lab_flat = labels.reshape(-1).astype(jnp.int32)
    per = _sc_losses(emb_flat, lab_flat).reshape(B, L)
    pull_loss = jnp.sum(per[:, 0]) / B
    push_loss = jnp.sum(per[:, 1]) / B
    return jnp.stack([pull_loss + push_loss, pull_loss, push_loss])


# speedup vs baseline: 1.7429x; 1.7429x over previous
import functools

import jax
import jax.numpy as jnp
from jax import lax
from jax.experimental import pallas as pl
from jax.experimental.pallas import tpu as pltpu
from jax.experimental.pallas import tpu_sc as plsc

B = 16
N = 4096
D = 16
K = 32
L = 16
CH = N // 2
DELTA_V = 0.1
TWO_DELTA_D = 1.0


def _rsqrt(x):
    i = plsc.bitcast(x, jnp.int32)
    y = plsc.bitcast(jnp.int32(0x5F3759DF) - (i >> 1), jnp.float32)
    for _ in range(3):
        y = y * (1.5 - 0.5 * x * y * y)
    return y


def _sqrt(x):
    return x * _rsqrt(x)


_GDN = lax.GatherDimensionNumbers(
    offset_dims=(), collapsed_slice_dims=(0,), start_index_map=(0,))


def _shuffle(x, idx):
    return lax.gather(x, idx[:, None], dimension_numbers=_GDN,
                      slice_sizes=(1,),
                      mode=lax.GatherScatterMode.PROMISE_IN_BOUNDS)


def _hsum(x, iota):
    for sh in (8, 4, 2, 1):
        x = x + _shuffle(x, iota ^ sh)
    return x


def _treduce(xs, iota):
    for s in (1, 2, 4, 8):
        msk = (iota & s) != 0
        xs = [jnp.where(msk, b, a) + _shuffle(jnp.where(msk, a, b), iota ^ s)
              for a, b in zip(xs[0::2], xs[1::2])]
    return xs[0]


def _sc_body(emb_hbm, lab_hbm, out_hbm,
             emb_v, lab_v, acc_v, part_v, pp_v, mean_v, mt_v, out_v, shared):
    c = lax.axis_index("c")
    s = lax.axis_index("s")
    batch = c * 8 + (s >> 1)
    half = s & 1
    base_pt = batch * N + half * CH

    with jax.named_scope("p0_dma_in"):
        pltpu.sync_copy(emb_hbm.at[pl.ds(base_pt * D, CH * D)], emb_v)
        pltpu.sync_copy(lab_hbm.at[pl.ds(base_pt, CH)], lab_v)

    zeros = jnp.zeros((L,), jnp.float32)
    ones = jnp.ones((L,), jnp.float32)
    iota16 = lax.iota(jnp.int32, L)

    def _zero(k, _):
        acc_v[pl.ds(k * L, L)] = zeros
        return 0
    lax.fori_loop(0, 96, _zero, 0)

    def _segsum(g, _):
        labs = lab_v[pl.ds(g * L, L)]
        for j in range(L):
            o = labs[j] * L
            row = emb_v[pl.ds((g * L + j) * L, L)]
            plsc.addupdate(acc_v.at[pl.ds(o, L)], row)
            plsc.addupdate(acc_v.at[pl.ds(512 + o, L)], ones)
        return 0
    with jax.named_scope("p1_segsum"):
        lax.fori_loop(0, CH // L, _segsum, 0)

    sh_base = s * 1536
    partner = (s ^ 1) * 1536
    pltpu.sync_copy(acc_v.at[pl.ds(0, 1024)], shared.at[pl.ds(sh_base, 1024)])
    plsc.subcore_barrier()
    pltpu.sync_copy(shared.at[pl.ds(partner, 1024)], part_v)

    def _comb(k, _):
        tot = acc_v[pl.ds(k * L, L)] + part_v[pl.ds(k * L, L)]
        cnt = acc_v[pl.ds(512 + k * L, L)] + part_v[pl.ds(512 + k * L, L)]
        acc_v[pl.ds(k * L, L)] = tot
        acc_v[pl.ds(512 + k * L, L)] = cnt
        mean_v[pl.ds(k * L, L)] = tot / jnp.maximum(cnt, 1.0)
        return 0
    with jax.named_scope("p2_combine"):
        lax.fori_loop(0, K, _comb, 0)

    def _pull(g, _):
        labs = lab_v[pl.ds(g * L, L)]
        sq = []
        for j in range(L):
            row = emb_v[pl.ds((g * L + j) * L, L)]
            dv = row - mean_v[pl.ds(labs[j] * L, L)]
            sq.append(dv * dv)
        y = jnp.maximum(_treduce(sq, iota16), 1e-20)
        dl = jnp.maximum(_sqrt(y) - DELTA_V, 0.0)
        plsc.addupdate_scatter(acc_v, [1024 + labs * L + iota16], dl)
        return 0
    with jax.named_scope("p3_pull"):
        lax.fori_loop(0, CH // L, _pull, 0)

    pltpu.sync_copy(acc_v.at[pl.ds(1024, 512)],
                    shared.at[pl.ds(sh_base + 1024, 512)])
    plsc.subcore_barrier()
    pltpu.sync_copy(shared.at[pl.ds(partner + 1024, 512)], pp_v)

    @pl.when(half == 0)
    def _finalize():
        def _combp(k, _):
            plsc.addupdate(acc_v.at[pl.ds(1024 + k * L, L)],
                           pp_v[pl.ds(k * L, L)])
            return 0
        lax.fori_loop(0, K, _combp, 0)

        diag = iota16 * 17
        cnt0 = plsc.load_gather(acc_v, [512 + diag])
        cnt1 = plsc.load_gather(acc_v, [768 + diag])
        ps0 = _treduce([acc_v[pl.ds(1024 + k * L, L)] for k in range(16)],
                       iota16)
        ps1 = _treduce([acc_v[pl.ds(1280 + k * L, L)] for k in range(16)],
                       iota16)
        f0 = jnp.where((cnt0 > 0.0) & (iota16 >= 1), 1.0, 0.0)
        f1 = jnp.where(cnt1 > 0.0, 1.0, 0.0)
        pim0 = ps0 / jnp.maximum(cnt0, 1.0)
        pim1 = ps1 / jnp.maximum(cnt1, 1.0)
        num_inst = _hsum(f0 + f1, iota16)
        instance_pull = _hsum(pim0 * f0 + pim1 * f1, iota16)
        pull_b = jnp.where(num_inst > 0.0,
                           instance_pull / (num_inst + 1e-6), 0.0)

        for k in range(K):
            plsc.store_scatter(mt_v, [iota16 * K + k], mean_v[pl.ds(k * L, L)])
        nsq0 = zeros
        nsq1 = zeros
        for d in range(D):
            t0 = mt_v[pl.ds(d * K, L)]
            t1 = mt_v[pl.ds(d * K + L, L)]
            nsq0 = nsq0 + t0 * t0
            nsq1 = nsq1 + t1 * t1
        rinv0 = jnp.minimum(_rsqrt(jnp.maximum(nsq0, 1e-30)), 1e12) * f0
        rinv1 = jnp.minimum(_rsqrt(jnp.maximum(nsq1, 1e-30)), 1e12) * f1
        for d in range(D):
            mt_v[pl.ds(d * K, L)] = mt_v[pl.ds(d * K, L)] * rinv0
            mt_v[pl.ds(d * K + L, L)] = mt_v[pl.ds(d * K + L, L)] * rinv1

        dsum = zeros
        msum = zeros
        for i in range(1, K):
            acc0 = zeros
            acc1 = zeros
            for d in range(D):
                mid = mt_v[pl.ds(d * K + (i // L) * L, L)][i % L]
                acc0 = acc0 + mid * mt_v[pl.ds(d * K, L)]
                acc1 = acc1 + mid * mt_v[pl.ds(d * K + L, L)]
            sqd0 = jnp.maximum(2.0 - 2.0 * acc0, 1e-20)
            sqd1 = jnp.maximum(2.0 - 2.0 * acc1, 1e-20)
            h0 = jnp.maximum(TWO_DELTA_D - _sqrt(sqd0), 0.0)
            h1 = jnp.maximum(TWO_DELTA_D - _sqrt(sqd1), 0.0)
            cnt_i = acc_v[pl.ds(512 + i * L, L)]
            fi = jnp.where(cnt_i > 0.0, 1.0, 0.0)
            m0 = f0 * fi * jnp.where(iota16 > i, 1.0, 0.0)
            m1 = f1 * fi * jnp.where(iota16 + L > i, 1.0, 0.0)
            dsum = dsum + h0 * m0 + h1 * m1
            msum = msum + m0 + m1
        push_b = jnp.where(num_inst > 1.0,
                           _hsum(dsum, iota16)
                           / (_hsum(msum, iota16) + 1e-6), 0.0)

        out_v[...] = (jnp.where(iota16 == 0, pull_b, 0.0)
                      + jnp.where(iota16 == 1, push_b, 0.0))
        pltpu.sync_copy(out_v, out_hbm.at[pl.ds(batch * L, L)])


@jax.jit
def _sc_losses(emb_flat, lab_flat):
    mesh = plsc.VectorSubcoreMesh(core_axis_name="c", subcore_axis_name="s")
    f = functools.partial(
        pl.kernel,
        mesh=mesh,
        compiler_params=pltpu.CompilerParams(needs_layout_passes=False),
        out_type=jax.ShapeDtypeStruct((B * L,), jnp.float32),
        scratch_types=[
            pltpu.VMEM((CH * D,), jnp.float32),
            pltpu.VMEM((CH,), jnp.int32),
            pltpu.VMEM((1536,), jnp.float32),
            pltpu.VMEM((1024,), jnp.float32),
            pltpu.VMEM((512,), jnp.float32),
            pltpu.VMEM((512,), jnp.float32),
            pltpu.VMEM((512,), jnp.float32),
            pltpu.VMEM((L,), jnp.float32),
            pltpu.VMEM_SHARED((16 * 1536,), jnp.float32),
        ],
    )(_sc_body)
    return f(emb_flat, lab_flat)


def kernel(embeddings, labels):
    emb_flat = embeddings.reshape(-1)
    lab_flat = labels.reshape(-1).astype(jnp.int32)
    per = _sc_losses(emb_flat, lab_flat).reshape(B, L)
    pull_loss = jnp.sum(per[:, 0]) / B
    push_loss = jnp.sum(per[:, 1]) / B
    return jnp.stack([pull_loss + push_loss, pull_loss, push_loss])


# speedup vs baseline: 1.9839x; 1.1382x over previous
import functools

import jax
import jax.numpy as jnp
from jax import lax
from jax.experimental import pallas as pl
from jax.experimental.pallas import tpu as pltpu
from jax.experimental.pallas import tpu_sc as plsc

B = 16
N = 4096
D = 16
K = 32
L = 16
CH = N // 2
DELTA_V = 0.1
TWO_DELTA_D = 1.0


def _rsqrt(x):
    i = plsc.bitcast(x, jnp.int32)
    y = plsc.bitcast(jnp.int32(0x5F3759DF) - (i >> 1), jnp.float32)
    for _ in range(3):
        y = y * (1.5 - 0.5 * x * y * y)
    return y


def _sqrt(x):
    return x * _rsqrt(x)


_GDN = lax.GatherDimensionNumbers(
    offset_dims=(), collapsed_slice_dims=(0,), start_index_map=(0,))


def _shuffle(x, idx):
    return lax.gather(x, idx[:, None], dimension_numbers=_GDN,
                      slice_sizes=(1,),
                      mode=lax.GatherScatterMode.PROMISE_IN_BOUNDS)


def _hsum(x, iota):
    for sh in (8, 4, 2, 1):
        x = x + _shuffle(x, iota ^ sh)
    return x


def _treduce(xs, iota):
    for s in (1, 2, 4, 8):
        msk = (iota & s) != 0
        xs = [jnp.where(msk, b, a) + _shuffle(jnp.where(msk, a, b), iota ^ s)
              for a, b in zip(xs[0::2], xs[1::2])]
    return xs[0]


def _sc_body(emb_hbm, lab_hbm, out_hbm,
             emb_v, lab_v, acc_v, part_v, pp_v, mean_v, mt_v, out_v, shared,
             sem_e, sem_l):
    c = lax.axis_index("c")
    s = lax.axis_index("s")
    batch = c * 8 + (s >> 1)
    half = s & 1
    base_pt = batch * N + half * CH

    with jax.named_scope("p0_dma_in"):
        cp_e = pltpu.async_copy(emb_hbm.at[pl.ds(base_pt * D, CH * D)],
                                emb_v, sem_e)
        cp_l = pltpu.async_copy(lab_hbm.at[pl.ds(base_pt, CH)], lab_v, sem_l)

        zeros = jnp.zeros((L,), jnp.float32)
        ones = jnp.ones((L,), jnp.float32)
        iota16 = lax.iota(jnp.int32, L)

        for k in range(96):
            acc_v[pl.ds(k * L, L)] = zeros
        cp_l.wait()
        cp_e.wait()

    @plsc.parallel_loop(0, CH // L, unroll=2)
    def _segsum(g):
        labs = lab_v[pl.ds(g * L, L)]
        plsc.addupdate_scatter(acc_v, [512 + labs * L + iota16], ones)
        for j in range(L):
            row = emb_v[pl.ds((g * L + j) * L, L)]
            plsc.addupdate(acc_v.at[pl.ds(labs[j] * L, L)], row)

    sh_base = s * 1536
    partner = (s ^ 1) * 1536
    with jax.named_scope("p1_segsum"):
        pltpu.sync_copy(acc_v.at[pl.ds(0, 1024)],
                        shared.at[pl.ds(sh_base, 1024)])
    plsc.subcore_barrier()
    with jax.named_scope("p2_combine"):
        pltpu.sync_copy(shared.at[pl.ds(partner, 1024)], part_v)
        for k in range(64):
            plsc.addupdate(acc_v.at[pl.ds(k * L, L)], part_v[pl.ds(k * L, L)])
        c0 = _treduce([acc_v[pl.ds(512 + k * L, L)] for k in range(16)],
                      iota16)
        c1 = _treduce([acc_v[pl.ds(768 + k * L, L)] for k in range(16)],
                      iota16)
        inv0 = 1.0 / jnp.maximum(c0, 1.0)
        inv1 = 1.0 / jnp.maximum(c1, 1.0)
        for k in range(K):
            inv = (inv0 if k < 16 else inv1)[k % L]
            mean_v[pl.ds(k * L, L)] = acc_v[pl.ds(k * L, L)] * inv

    @plsc.parallel_loop(0, CH // L, unroll=2)
    def _pull(g):
        labs = lab_v[pl.ds(g * L, L)]
        sq = []
        for j in range(L):
            row = emb_v[pl.ds((g * L + j) * L, L)]
            dv = row - mean_v[pl.ds(labs[j] * L, L)]
            sq.append(dv * dv)
        y = jnp.maximum(_treduce(sq, iota16), 1e-20)
        dl = jnp.maximum(_sqrt(y) - DELTA_V, 0.0)
        plsc.addupdate_scatter(acc_v, [1024 + labs * L + iota16], dl)

    with jax.named_scope("p3_pull"):
        pltpu.sync_copy(acc_v.at[pl.ds(1024, 512)],
                        shared.at[pl.ds(sh_base + 1024, 512)])
    plsc.subcore_barrier()
    pltpu.sync_copy(shared.at[pl.ds(partner + 1024, 512)], pp_v)

    @pl.when(half == 0)
    def _finalize():
        def _combp(k, _):
            plsc.addupdate(acc_v.at[pl.ds(1024 + k * L, L)],
                           pp_v[pl.ds(k * L, L)])
            return 0
        lax.fori_loop(0, K, _combp, 0)

        ps0 = _treduce([acc_v[pl.ds(1024 + k * L, L)] for k in range(16)],
                       iota16)
        ps1 = _treduce([acc_v[pl.ds(1280 + k * L, L)] for k in range(16)],
                       iota16)
        f0 = jnp.where((c0 > 0.0) & (iota16 >= 1), 1.0, 0.0)
        f1 = jnp.where(c1 > 0.0, 1.0, 0.0)
        pim0 = ps0 * inv0
        pim1 = ps1 * inv1
        num_inst = _hsum(f0 + f1, iota16)
        instance_pull = _hsum(pim0 * f0 + pim1 * f1, iota16)
        pull_b = jnp.where(num_inst > 0.0,
                           instance_pull / (num_inst + 1e-6), 0.0)

        for k in range(K):
            plsc.store_scatter(mt_v, [iota16 * K + k], mean_v[pl.ds(k * L, L)])
        nsq0 = zeros
        nsq1 = zeros
        for d in range(D):
            t0 = mt_v[pl.ds(d * K, L)]
            t1 = mt_v[pl.ds(d * K + L, L)]
            nsq0 = nsq0 + t0 * t0
            nsq1 = nsq1 + t1 * t1
        rinv0 = jnp.minimum(_rsqrt(jnp.maximum(nsq0, 1e-30)), 1e12) * f0
        rinv1 = jnp.minimum(_rsqrt(jnp.maximum(nsq1, 1e-30)), 1e12) * f1
        for d in range(D):
            mt_v[pl.ds(d * K, L)] = mt_v[pl.ds(d * K, L)] * rinv0
            mt_v[pl.ds(d * K + L, L)] = mt_v[pl.ds(d * K + L, L)] * rinv1

        dsum = zeros
        msum = zeros
        for i in range(1, K):
            acc0 = zeros
            acc1 = zeros
            for d in range(D):
                mid = mt_v[pl.ds(d * K + (i // L) * L, L)][i % L]
                acc0 = acc0 + mid * mt_v[pl.ds(d * K, L)]
                acc1 = acc1 + mid * mt_v[pl.ds(d * K + L, L)]
            sqd0 = jnp.maximum(2.0 - 2.0 * acc0, 1e-20)
            sqd1 = jnp.maximum(2.0 - 2.0 * acc1, 1e-20)
            h0 = jnp.maximum(TWO_DELTA_D - _sqrt(sqd0), 0.0)
            h1 = jnp.maximum(TWO_DELTA_D - _sqrt(sqd1), 0.0)
            fi = (f0 if i < 16 else f1)[i % L]
            m0 = f0 * fi * jnp.where(iota16 > i, 1.0, 0.0)
            m1 = f1 * fi * jnp.where(iota16 + L > i, 1.0, 0.0)
            dsum = dsum + h0 * m0 + h1 * m1
            msum = msum + m0 + m1
        push_b = jnp.where(num_inst > 1.0,
                           _hsum(dsum, iota16)
                           / (_hsum(msum, iota16) + 1e-6), 0.0)

        out_v[...] = (jnp.where(iota16 == 0, pull_b, 0.0)
                      + jnp.where(iota16 == 1, push_b, 0.0))
        pltpu.sync_copy(out_v, out_hbm.at[pl.ds(batch * L, L)])


@jax.jit
def _sc_losses(emb_flat, lab_flat):
    mesh = plsc.VectorSubcoreMesh(core_axis_name="c", subcore_axis_name="s")
    f = functools.partial(
        pl.kernel,
        mesh=mesh,
        compiler_params=pltpu.CompilerParams(needs_layout_passes=False),
        out_type=jax.ShapeDtypeStruct((B * L,), jnp.float32),
        scratch_types=[
            pltpu.VMEM((CH * D,), jnp.float32),
            pltpu.VMEM((CH,), jnp.int32),
            pltpu.VMEM((1536,), jnp.float32),
            pltpu.VMEM((1024,), jnp.float32),
            pltpu.VMEM((512,), jnp.float32),
            pltpu.VMEM((512,), jnp.float32),
            pltpu.VMEM((512,), jnp.float32),
            pltpu.VMEM((L,), jnp.float32),
            pltpu.VMEM_SHARED((16 * 1536,), jnp.float32),
            pltpu.SemaphoreType.DMA,
            pltpu.SemaphoreType.DMA,
        ],
    )(_sc_body)
    return f(emb_flat, lab_flat)


def kernel(embeddings, labels):
    emb_flat = embeddings.reshape(-1)
    lab_flat = labels.reshape(-1).astype(jnp.int32)
    per = _sc_losses(emb_flat, lab_flat).reshape(B, L)
    pull_loss = jnp.sum(per[:, 0]) / B
    push_loss = jnp.sum(per[:, 1]) / B
    return jnp.stack([pull_loss + push_loss, pull_loss, push_loss])
